# baseline (speedup 1.0000x reference)
import functools

import jax
import jax.numpy as jnp
from jax import lax
from jax.experimental import pallas as pl
from jax.experimental.pallas import tpu as pltpu

_N = 320000
_C = 128
_S = 64
_EPS = 1e-05
_B = 3200
_NB = _N // _B


def _stats_body(x_ref, seg_ref, w_ref, b_ref, out_ref, acc_ref):
    i = pl.program_id(0)

    @pl.when(i == 0)
    def _init():
        acc_ref[...] = jnp.zeros_like(acc_ref)

    x = x_ref[...]
    segs = seg_ref[0, 0, :]
    onehot_t = (lax.broadcasted_iota(jnp.int32, (_S, _B), 0)
                == segs[None, :]).astype(jnp.bfloat16)
    y = jnp.concatenate([x, x * x, jnp.ones_like(x)],
                        axis=1).astype(jnp.bfloat16)
    acc_ref[...] += jnp.dot(onehot_t, y, preferred_element_type=jnp.float32)

    @pl.when(i == _NB - 1)
    def _finalize():
        acc = acc_ref[...]
        sums = acc[:, :_C]
        sumsq = acc[:, _C:2 * _C]
        cnt = jnp.maximum(acc[:, 2 * _C:], 1.0)
        mean = sums / cnt
        var = jnp.maximum(sumsq / cnt - mean * mean, 0.0)
        instd = lax.rsqrt(var + _EPS)
        scale = instd * w_ref[0, :][None, :]
        shift = b_ref[0, :][None, :] - mean * scale
        out_ref[...] = jnp.concatenate([scale, shift], axis=1)


def _apply_body(x_ref, seg_ref, st_ref, o_ref):
    segs = seg_ref[0, 0, :]
    onehot = (segs[:, None]
              == lax.broadcasted_iota(jnp.int32, (_B, _S), 1)).astype(jnp.bfloat16)
    rows = jnp.dot(onehot, st_ref[...].astype(jnp.bfloat16),
                   preferred_element_type=jnp.float32)
    o_ref[...] = x_ref[...] * rows[:, :_C] + rows[:, _C:]


@jax.jit
def kernel(x, segment_ids, weight, bias):
    seg3 = segment_ids.astype(jnp.int32).reshape(_NB, 1, _B)

    stats = pl.pallas_call(
        _stats_body,
        grid=(_NB,),
        in_specs=[
            pl.BlockSpec((_B, _C), lambda i: (i, 0)),
            pl.BlockSpec((1, 1, _B), lambda i: (i, 0, 0)),
            pl.BlockSpec((1, _C), lambda i: (0, 0)),
            pl.BlockSpec((1, _C), lambda i: (0, 0)),
        ],
        out_specs=pl.BlockSpec((_S, 2 * _C), lambda i: (0, 0)),
        out_shape=jax.ShapeDtypeStruct((_S, 2 * _C), jnp.float32),
        scratch_shapes=[pltpu.VMEM((_S, 3 * _C), jnp.float32)],
    )(x, seg3, weight, bias)

    out = pl.pallas_call(
        _apply_body,
        grid=(_NB,),
        in_specs=[
            pl.BlockSpec((_B, _C), lambda i: (i, 0)),
            pl.BlockSpec((1, 1, _B), lambda i: (i, 0, 0)),
            pl.BlockSpec((_S, 2 * _C), lambda i: (0, 0)),
        ],
        out_specs=pl.BlockSpec((_B, _C), lambda i: (i, 0)),
        out_shape=jax.ShapeDtypeStruct((_N, _C), jnp.float32),
    )(x, seg3, stats)
    return out


# speedup vs baseline: 1.3069x; 1.3069x over previous
import functools

import jax
import jax.numpy as jnp
from jax import lax
from jax.experimental import pallas as pl
from jax.experimental.pallas import tpu as pltpu

_N = 320000
_C = 128
_S = 64
_EPS = 1e-05
_B = 6400
_NB = _N // _B


def _stats_body(x_ref, seg_ref, w_ref, b_ref, out_ref, acc_ref):
    i = pl.program_id(0)

    @pl.when(i == 0)
    def _init():
        acc_ref[...] = jnp.zeros_like(acc_ref)

    x = x_ref[...]
    segs = seg_ref[0, 0, :]
    onehot_t = (lax.broadcasted_iota(jnp.int32, (_S, _B), 0)
                == segs[None, :]).astype(jnp.bfloat16)
    y = jnp.concatenate([x, x * x, jnp.ones_like(x)],
                        axis=1).astype(jnp.bfloat16)
    acc_ref[...] += jnp.dot(onehot_t, y, preferred_element_type=jnp.float32)

    @pl.when(i == _NB - 1)
    def _finalize():
        acc = acc_ref[...]
        sums = acc[:, :_C]
        sumsq = acc[:, _C:2 * _C]
        cnt = jnp.maximum(acc[:, 2 * _C:], 1.0)
        mean = sums / cnt
        var = jnp.maximum(sumsq / cnt - mean * mean, 0.0)
        instd = lax.rsqrt(var + _EPS)
        scale = instd * w_ref[0, :][None, :]
        shift = b_ref[0, :][None, :] - mean * scale
        out_ref[...] = jnp.concatenate([scale, shift], axis=1)


def _apply_body(x_ref, seg_ref, st_ref, o_ref):
    segs = seg_ref[0, 0, :]
    onehot = (segs[:, None]
              == lax.broadcasted_iota(jnp.int32, (_B, _S), 1)).astype(jnp.bfloat16)
    rows = jnp.dot(onehot, st_ref[...].astype(jnp.bfloat16),
                   preferred_element_type=jnp.float32)
    o_ref[...] = x_ref[...] * rows[:, :_C] + rows[:, _C:]


@jax.jit
def kernel(x, segment_ids, weight, bias):
    seg3 = segment_ids.astype(jnp.int32).reshape(_NB, 1, _B)

    stats = pl.pallas_call(
        _stats_body,
        grid=(_NB,),
        in_specs=[
            pl.BlockSpec((_B, _C), lambda i: (i, 0)),
            pl.BlockSpec((1, 1, _B), lambda i: (i, 0, 0)),
            pl.BlockSpec((1, _C), lambda i: (0, 0)),
            pl.BlockSpec((1, _C), lambda i: (0, 0)),
        ],
        out_specs=pl.BlockSpec((_S, 2 * _C), lambda i: (0, 0)),
        out_shape=jax.ShapeDtypeStruct((_S, 2 * _C), jnp.float32),
        scratch_shapes=[pltpu.VMEM((_S, 3 * _C), jnp.float32)],
    )(x, seg3, weight, bias)

    out = pl.pallas_call(
        _apply_body,
        grid=(_NB,),
        in_specs=[
            pl.BlockSpec((_B, _C), lambda i: (i, 0)),
            pl.BlockSpec((1, 1, _B), lambda i: (i, 0, 0)),
            pl.BlockSpec((_S, 2 * _C), lambda i: (0, 0)),
        ],
        out_specs=pl.BlockSpec((_B, _C), lambda i: (i, 0)),
        out_shape=jax.ShapeDtypeStruct((_N, _C), jnp.float32),
    )(x, seg3, stats)
    return out


# speedup vs baseline: 1.4794x; 1.1320x over previous
import functools

import jax
import jax.numpy as jnp
from jax import lax
from jax.experimental import pallas as pl
from jax.experimental.pallas import tpu as pltpu

_N = 320000
_C = 128
_S = 64
_EPS = 1e-05
_B = 12800
_NB = _N // _B


def _stats_body(x_ref, seg_ref, w_ref, b_ref, out_ref, acc_ref):
    i = pl.program_id(0)

    @pl.when(i == 0)
    def _init():
        acc_ref[...] = jnp.zeros_like(acc_ref)

    x = x_ref[...]
    segs = seg_ref[0, 0, :]
    onehot_t = (lax.broadcasted_iota(jnp.int32, (_S, _B), 0)
                == segs[None, :]).astype(jnp.bfloat16)
    y = jnp.concatenate([x, x * x, jnp.ones_like(x)],
                        axis=1).astype(jnp.bfloat16)
    acc_ref[...] += jnp.dot(onehot_t, y, preferred_element_type=jnp.float32)

    @pl.when(i == _NB - 1)
    def _finalize():
        acc = acc_ref[...]
        sums = acc[:, :_C]
        sumsq = acc[:, _C:2 * _C]
        cnt = jnp.maximum(acc[:, 2 * _C:], 1.0)
        mean = sums / cnt
        var = jnp.maximum(sumsq / cnt - mean * mean, 0.0)
        instd = lax.rsqrt(var + _EPS)
        scale = instd * w_ref[0, :][None, :]
        shift = b_ref[0, :][None, :] - mean * scale
        out_ref[...] = jnp.concatenate([scale, shift], axis=1)


def _apply_body(x_ref, seg_ref, st_ref, o_ref):
    segs = seg_ref[0, 0, :]
    onehot = (segs[:, None]
              == lax.broadcasted_iota(jnp.int32, (_B, _S), 1)).astype(jnp.bfloat16)
    rows = jnp.dot(onehot, st_ref[...].astype(jnp.bfloat16),
                   preferred_element_type=jnp.float32)
    o_ref[...] = x_ref[...] * rows[:, :_C] + rows[:, _C:]


@jax.jit
def kernel(x, segment_ids, weight, bias):
    seg3 = segment_ids.astype(jnp.int32).reshape(_NB, 1, _B)

    stats = pl.pallas_call(
        _stats_body,
        grid=(_NB,),
        in_specs=[
            pl.BlockSpec((_B, _C), lambda i: (i, 0)),
            pl.BlockSpec((1, 1, _B), lambda i: (i, 0, 0)),
            pl.BlockSpec((1, _C), lambda i: (0, 0)),
            pl.BlockSpec((1, _C), lambda i: (0, 0)),
        ],
        out_specs=pl.BlockSpec((_S, 2 * _C), lambda i: (0, 0)),
        out_shape=jax.ShapeDtypeStruct((_S, 2 * _C), jnp.float32),
        scratch_shapes=[pltpu.VMEM((_S, 3 * _C), jnp.float32)],
    )(x, seg3, weight, bias)

    out = pl.pallas_call(
        _apply_body,
        grid=(_NB,),
        in_specs=[
            pl.BlockSpec((_B, _C), lambda i: (i, 0)),
            pl.BlockSpec((1, 1, _B), lambda i: (i, 0, 0)),
            pl.BlockSpec((_S, 2 * _C), lambda i: (0, 0)),
        ],
        out_specs=pl.BlockSpec((_B, _C), lambda i: (i, 0)),
        out_shape=jax.ShapeDtypeStruct((_N, _C), jnp.float32),
    )(x, seg3, stats)
    return out


# speedup vs baseline: 1.5387x; 1.0401x over previous
import functools

import jax
import jax.numpy as jnp
from jax import lax
from jax.experimental import pallas as pl
from jax.experimental.pallas import tpu as pltpu

_N = 320000
_C = 128
_S = 64
_EPS = 1e-05
_B = 20000
_NB = _N // _B


def _stats_body(x_ref, seg_ref, w_ref, b_ref, out_ref, acc_ref):
    i = pl.program_id(0)

    @pl.when(i == 0)
    def _init():
        acc_ref[...] = jnp.zeros_like(acc_ref)

    x = x_ref[...]
    segs = seg_ref[0, 0, :]
    onehot_t = (lax.broadcasted_iota(jnp.int32, (_S, _B), 0)
                == segs[None, :]).astype(jnp.bfloat16)
    y = jnp.concatenate([x, x * x, jnp.ones_like(x)],
                        axis=1).astype(jnp.bfloat16)
    acc_ref[...] += jnp.dot(onehot_t, y, preferred_element_type=jnp.float32)

    @pl.when(i == _NB - 1)
    def _finalize():
        acc = acc_ref[...]
        sums = acc[:, :_C]
        sumsq = acc[:, _C:2 * _C]
        cnt = jnp.maximum(acc[:, 2 * _C:], 1.0)
        mean = sums / cnt
        var = jnp.maximum(sumsq / cnt - mean * mean, 0.0)
        instd = lax.rsqrt(var + _EPS)
        scale = instd * w_ref[0, :][None, :]
        shift = b_ref[0, :][None, :] - mean * scale
        out_ref[...] = jnp.concatenate([scale, shift], axis=1)


def _apply_body(x_ref, seg_ref, st_ref, o_ref):
    segs = seg_ref[0, 0, :]
    onehot = (segs[:, None]
              == lax.broadcasted_iota(jnp.int32, (_B, _S), 1)).astype(jnp.bfloat16)
    rows = jnp.dot(onehot, st_ref[...].astype(jnp.bfloat16),
                   preferred_element_type=jnp.float32)
    o_ref[...] = x_ref[...] * rows[:, :_C] + rows[:, _C:]


@jax.jit
def kernel(x, segment_ids, weight, bias):
    seg3 = segment_ids.astype(jnp.int32).reshape(_NB, 1, _B)

    stats = pl.pallas_call(
        _stats_body,
        grid=(_NB,),
        in_specs=[
            pl.BlockSpec((_B, _C), lambda i: (i, 0)),
            pl.BlockSpec((1, 1, _B), lambda i: (i, 0, 0)),
            pl.BlockSpec((1, _C), lambda i: (0, 0)),
            pl.BlockSpec((1, _C), lambda i: (0, 0)),
        ],
        out_specs=pl.BlockSpec((_S, 2 * _C), lambda i: (0, 0)),
        out_shape=jax.ShapeDtypeStruct((_S, 2 * _C), jnp.float32),
        scratch_shapes=[pltpu.VMEM((_S, 3 * _C), jnp.float32)],
    )(x, seg3, weight, bias)

    out = pl.pallas_call(
        _apply_body,
        grid=(_NB,),
        in_specs=[
            pl.BlockSpec((_B, _C), lambda i: (i, 0)),
            pl.BlockSpec((1, 1, _B), lambda i: (i, 0, 0)),
            pl.BlockSpec((_S, 2 * _C), lambda i: (0, 0)),
        ],
        out_specs=pl.BlockSpec((_B, _C), lambda i: (i, 0)),
        out_shape=jax.ShapeDtypeStruct((_N, _C), jnp.float32),
    )(x, seg3, stats)
    return out
